# baseline (speedup 1.0000x reference)
import jax
import jax.numpy as jnp
from jax import lax
from jax.experimental import pallas as pl
from jax.experimental.pallas import tpu as pltpu
from jax.experimental.pallas import tpu_sc as plsc

A = 26
V = 100000
D = 64
B = 16384
TOTAL = B * A

NC, NS = 2, 16
NW = NC * NS

IDXW = 128
CHUNK = 512
NJ = CHUNK // IDXW
NCHUNK = TOTAL // NW // CHUNK


def _gather_body(idx_hbm, didx_hbm, tab_hbm, out_hbm, idx_v, didx_v, rows_v,
                 gsems, wsems):
    wid = lax.axis_index("s") * NC + lax.axis_index("c")
    NS_ = 3

    def stage(c):
        s = c % NS_
        base = wid * NCHUNK + c
        pltpu.sync_copy(idx_hbm.at[base], idx_v.at[s])
        pltpu.sync_copy(didx_hbm.at[base], didx_v.at[s])
        for j in range(NJ):
            pltpu.async_copy(
                tab_hbm.at[idx_v.at[s, pl.ds(j * IDXW, IDXW)]],
                rows_v.at[s, pl.ds(j * IDXW, IDXW)],
                gsems.at[s],
            )

    def wait_gathers(c):
        s = c % NS_
        for j in range(NJ):
            pltpu.make_async_copy(
                tab_hbm.at[idx_v.at[s, pl.ds(j * IDXW, IDXW)]],
                rows_v.at[s, pl.ds(j * IDXW, IDXW)],
                gsems.at[s],
            ).wait()

    def fire_writes(c):
        s = c % NS_
        for j in range(NJ):
            pltpu.async_copy(
                rows_v.at[s, pl.ds(j * IDXW, IDXW)],
                out_hbm.at[didx_v.at[s, j]],
                wsems.at[s],
            )

    def wait_writes(c):
        s = c % NS_
        for j in range(NJ):
            pltpu.make_async_copy(
                rows_v.at[s, pl.ds(j * IDXW, IDXW)],
                out_hbm.at[didx_v.at[s, j]],
                wsems.at[s],
            ).wait()

    for c in range(NCHUNK):
        if c >= NS_:
            wait_writes(c - NS_)
        stage(c)
        if c >= 1:
            wait_gathers(c - 1)
            fire_writes(c - 1)
    wait_gathers(NCHUNK - 1)
    fire_writes(NCHUNK - 1)
    for c in range(NCHUNK - NS_ + 1, NCHUNK):
        wait_writes(c - 1)
    wait_writes(NCHUNK - 1)


def _gather(flat_idx, dst_idx, unit_tab):
    mesh = plsc.VectorSubcoreMesh(core_axis_name="c", subcore_axis_name="s")
    f = pl.kernel(
        _gather_body,
        out_type=jax.ShapeDtypeStruct((TOTAL, D), jnp.float32),
        mesh=mesh,
        scratch_types=[
            pltpu.VMEM((3, CHUNK), jnp.int32),
            pltpu.VMEM((3, NJ, IDXW), jnp.int32),
            pltpu.VMEM((3, CHUNK, D), jnp.float32),
            pltpu.SemaphoreType.DMA((3,)),
            pltpu.SemaphoreType.DMA((3,)),
        ],
        compiler_params=pltpu.CompilerParams(
            use_tc_tiling_on_sc=False, needs_layout_passes=False
        ),
    )
    return f(flat_idx, dst_idx, unit_tab)


def kernel(tuples, tables):
    offs = (jnp.arange(A, dtype=jnp.int32) * (2 * V))[None, :]
    flat_idx = (2 * tuples + offs).reshape(TOTAL // CHUNK, CHUNK)
    unit_tab = jnp.concatenate([tables, jnp.zeros_like(tables)], axis=2).reshape(2 * A * V, D)
    r = jnp.arange(TOTAL, dtype=jnp.int32)
    b, a = r // A, r % A
    dst = (b >> 3) * (16 * (A // 2)) + (a >> 1) * 16 + (b & 7) * 2 + (a & 1)
    dst_idx = dst.reshape(TOTAL // CHUNK, NJ, IDXW)
    out = _gather(flat_idx, dst_idx, unit_tab)
    y = out.reshape(B // 8, A // 2, 8, 2 * D)
    return y.transpose(0, 2, 1, 3).reshape(B, A * D)


# speedup vs baseline: 1.0009x; 1.0009x over previous
import jax
import jax.numpy as jnp
from jax import lax
from jax.experimental import pallas as pl
from jax.experimental.pallas import tpu as pltpu
from jax.experimental.pallas import tpu_sc as plsc

A = 26
V = 100000
D = 64
B = 16384
TOTAL = B * A

NC, NS = 2, 16
NW = NC * NS

IDXW = 128
CHUNK = 512
NJ = CHUNK // IDXW
NCHUNK = TOTAL // NW // CHUNK


def _gather_body(idx_hbm, didx_hbm, tab_hbm, out_hbm, idx_v, didx_v, rows_v,
                 gsems, wsems):
    wid = lax.axis_index("s") * NC + lax.axis_index("c")
    NS_ = 3

    def stage(c):
        s = c % NS_
        base = wid * NCHUNK + c
        pltpu.sync_copy(idx_hbm.at[base], idx_v.at[s])
        pltpu.sync_copy(didx_hbm.at[base], didx_v.at[s])
        for j in range(NJ):
            pltpu.async_copy(
                tab_hbm.at[idx_v.at[s, pl.ds(j * IDXW, IDXW)]],
                rows_v.at[s, pl.ds(j * IDXW, IDXW)],
                gsems.at[s],
            )

    def wait_gathers(c):
        s = c % NS_
        for j in range(NJ):
            pltpu.make_async_copy(
                tab_hbm.at[idx_v.at[s, pl.ds(j * IDXW, IDXW)]],
                rows_v.at[s, pl.ds(j * IDXW, IDXW)],
                gsems.at[s],
            ).wait()

    def fire_writes(c):
        s = c % NS_
        for j in range(NJ):
            pltpu.async_copy(
                rows_v.at[s, pl.ds(j * IDXW, IDXW)],
                out_hbm.at[didx_v.at[s, j]],
                wsems.at[s],
            )

    def wait_writes(c):
        s = c % NS_
        for j in range(NJ):
            pltpu.make_async_copy(
                rows_v.at[s, pl.ds(j * IDXW, IDXW)],
                out_hbm.at[didx_v.at[s, j]],
                wsems.at[s],
            ).wait()

    for c in range(NCHUNK):
        if c >= NS_:
            wait_writes(c - NS_)
        stage(c)
        if c >= 1:
            wait_gathers(c - 1)
            fire_writes(c - 1)
    wait_gathers(NCHUNK - 1)
    fire_writes(NCHUNK - 1)
    for c in range(NCHUNK - NS_ + 1, NCHUNK):
        wait_writes(c - 1)
    wait_writes(NCHUNK - 1)


def _gather(flat_idx, dst_idx, unit_tab):
    mesh = plsc.VectorSubcoreMesh(core_axis_name="c", subcore_axis_name="s")
    f = pl.kernel(
        _gather_body,
        out_type=jax.ShapeDtypeStruct((TOTAL, D), jnp.float32),
        mesh=mesh,
        scratch_types=[
            pltpu.VMEM((3, CHUNK), jnp.int32),
            pltpu.VMEM((3, NJ, IDXW), jnp.int32),
            pltpu.VMEM((3, CHUNK, D), jnp.float32),
            pltpu.SemaphoreType.DMA((3,)),
            pltpu.SemaphoreType.DMA((3,)),
        ],
        compiler_params=pltpu.CompilerParams(
            use_tc_tiling_on_sc=False, needs_layout_passes=False
        ),
    )
    return f(flat_idx, dst_idx, unit_tab)


def kernel(tuples, tables):
    offs = (jnp.arange(A, dtype=jnp.int32) * (2 * V))[None, :]
    flat_idx = (2 * tuples + offs).reshape(TOTAL // CHUNK, CHUNK)
    unit_tab = jnp.pad(tables, ((0, 0), (0, 0), (0, D))).reshape(2 * A * V, D)
    r = jnp.arange(TOTAL, dtype=jnp.int32)
    b, a = r // A, r % A
    dst = (b >> 3) * (16 * (A // 2)) + (a >> 1) * 16 + (b & 7) * 2 + (a & 1)
    dst_idx = dst.reshape(TOTAL // CHUNK, NJ, IDXW)
    out = _gather(flat_idx, dst_idx, unit_tab)
    y = out.reshape(B // 8, A // 2, 8, 2 * D)
    return y.transpose(0, 2, 1, 3).reshape(B, A * D)
